# baseline (speedup 1.0000x reference)
import functools

import jax
import jax.numpy as jnp
from jax import lax
from jax.experimental import pallas as pl
from jax.experimental.pallas import tpu as pltpu
from jax.experimental.pallas import tpu_sc as plsc

NC = 2
NS = 16
NW = NC * NS
CHUNK = 128


def _mesh():
    return plsc.VectorSubcoreMesh(
        core_axis_name="c", subcore_axis_name="s", num_cores=NC, num_subcores=NS
    )


def _sc_gather(tab, row, col):
    n, h = tab.shape
    hw = h // 2
    e = row.shape[0]
    nch = e // CHUNK
    slots = -(-nch // NW)
    npairs = -(-slots // 2) * 2

    def body(tab_hbm, row_hbm, col_hbm, g_hbm, idxr, idxc, bufr, bufc, sems):
        cid = lax.axis_index("c")
        sid = lax.axis_index("s")
        wid = sid * NC + cid

        def start(j, b):
            ch = j * NW + wid

            @pl.when(ch < nch)
            def _():
                base = ch * CHUNK
                pltpu.sync_copy(row_hbm.at[pl.ds(base, CHUNK)], idxr.at[b])
                pltpu.sync_copy(col_hbm.at[pl.ds(base, CHUNK)], idxc.at[b])
                pltpu.async_copy(tab_hbm.at[idxr.at[b]], bufr.at[b], sems.at[2 * b])
                pltpu.async_copy(tab_hbm.at[idxc.at[b]], bufc.at[b], sems.at[2 * b + 1])

        def drain(j, b):
            ch = j * NW + wid

            @pl.when(ch < nch)
            def _():
                base = ch * CHUNK
                pltpu.make_async_copy(tab_hbm.at[idxr.at[b]], bufr.at[b],
                                      sems.at[2 * b]).wait()
                pltpu.make_async_copy(tab_hbm.at[idxc.at[b]], bufc.at[b],
                                      sems.at[2 * b + 1]).wait()

                def splice(i, carry):
                    for k in range(hw // 16):
                        bufr[b, i, pl.ds(hw + k * 16, 16)] = (
                            bufc[b, i, pl.ds(hw + k * 16, 16)])
                    return carry

                lax.fori_loop(0, CHUNK, splice, 0)
                pltpu.sync_copy(bufr.at[b], g_hbm.at[pl.ds(base, CHUNK)])

        start(0, 0)

        def step(jj, carry):
            j0 = jj * 2
            start(j0 + 1, 1)
            drain(j0, 0)
            start(j0 + 2, 0)
            drain(j0 + 1, 1)
            return carry

        lax.fori_loop(0, npairs // 2, step, 0)

    f = pl.kernel(
        body,
        out_type=jax.ShapeDtypeStruct((e, h), jnp.float32),
        mesh=_mesh(),
        scratch_types=[
            pltpu.VMEM((2, CHUNK), jnp.int32),
            pltpu.VMEM((2, CHUNK), jnp.int32),
            pltpu.VMEM((2, CHUNK, h), jnp.float32),
            pltpu.VMEM((2, CHUNK, h), jnp.float32),
            pltpu.SemaphoreType.DMA((4,)),
        ],
    )
    return f(tab, row, col)


def _sc_scatter(m2, row, zeros_nh):
    e, h = m2.shape
    n = zeros_nh.shape[0]
    nch = e // CHUNK
    slots = -(-nch // NW)
    npairs = -(-slots // 2) * 2
    rq = (n // NS) // 8 * 8
    tail_off = NS * rq
    tail = n - tail_off

    def body(m2_hbm, row_hbm, z_hbm, agg_hbm, idx, buf, acc, sems):
        cid = lax.axis_index("c")
        sid = lax.axis_index("s")
        wid = sid * NC + cid

        pltpu.sync_copy(z_hbm.at[pl.ds(sid * rq, rq)],
                        acc.at[pl.ds(sid * rq, rq)])
        if tail:
            @pl.when(sid == 0)
            def _():
                pltpu.sync_copy(z_hbm.at[pl.ds(tail_off, tail)],
                                acc.at[pl.ds(tail_off, tail)])
        plsc.subcore_barrier()

        def load(j, b):
            ch = j * NW + wid

            @pl.when(ch < nch)
            def _():
                base = ch * CHUNK
                pltpu.sync_copy(row_hbm.at[pl.ds(base, CHUNK)], idx.at[b])
                pltpu.async_copy(m2_hbm.at[pl.ds(base, CHUNK)], buf.at[b],
                                 sems.at[b])

        def flush(j, b):
            ch = j * NW + wid

            @pl.when(ch < nch)
            def _():
                base = ch * CHUNK
                pltpu.make_async_copy(m2_hbm.at[pl.ds(base, CHUNK)],
                                      buf.at[b], sems.at[b]).wait()
                pltpu.sync_copy(buf.at[b], acc.at[idx.at[b]], add=True)

        load(0, 0)

        def step(jj, carry):
            j0 = jj * 2
            load(j0 + 1, 1)
            flush(j0, 0)
            load(j0 + 2, 0)
            flush(j0 + 1, 1)
            return carry

        lax.fori_loop(0, npairs // 2, step, 0)
        plsc.subcore_barrier()
        pltpu.sync_copy(acc.at[pl.ds(sid * rq, rq)],
                        agg_hbm.at[cid, pl.ds(sid * rq, rq)])
        if tail:
            @pl.when(sid == 0)
            def _():
                pltpu.sync_copy(acc.at[pl.ds(tail_off, tail)],
                                agg_hbm.at[cid, pl.ds(tail_off, tail)])

    f = pl.kernel(
        body,
        out_type=jax.ShapeDtypeStruct((NC, n, h), jnp.float32),
        mesh=_mesh(),
        scratch_types=[
            pltpu.VMEM((2, CHUNK), jnp.int32),
            pltpu.VMEM((2, CHUNK, h), jnp.float32),
            pltpu.VMEM_SHARED((n, h), jnp.float32),
            pltpu.SemaphoreType.DMA((2,)),
        ],
    )
    return f(m2, row, zeros_nh)


def _silu(x):
    return x * jax.nn.sigmoid(x)


def _row_specs(rt, h, n_weights, n_biases):
    in_specs = [pl.BlockSpec((rt, h), lambda i: (i, 0))]
    in_specs += [pl.BlockSpec((h, h), lambda i: (0, 0))] * n_weights
    in_specs += [pl.BlockSpec((1, h), lambda i: (0, 0))] * n_biases
    return in_specs


def _embed_body(h_ref, w_ref, b_ref, o_ref):
    o_ref[...] = jnp.dot(h_ref[...], w_ref[...],
                         preferred_element_type=jnp.float32) + b_ref[...]


def _embed(x, w, b, rt=1000):
    n, h = x.shape
    return pl.pallas_call(
        _embed_body,
        grid=(n // rt,),
        in_specs=_row_specs(rt, h, 1, 1),
        out_specs=pl.BlockSpec((rt, h), lambda i: (i, 0)),
        out_shape=jax.ShapeDtypeStruct((n, w.shape[1]), jnp.float32),
    )(x, w, b)


def _pack16(lo, hi):
    lo_u = jax.lax.bitcast_convert_type(lo.astype(jnp.bfloat16), jnp.uint16)
    hi_u = jax.lax.bitcast_convert_type(hi.astype(jnp.bfloat16), jnp.uint16)
    w = lo_u.astype(jnp.uint32) | (hi_u.astype(jnp.uint32) << 16)
    return jax.lax.bitcast_convert_type(w, jnp.float32)


def _unpack16(w):
    u = jax.lax.bitcast_convert_type(w, jnp.uint32)
    lo = jax.lax.bitcast_convert_type((u & 0xFFFF).astype(jnp.uint16),
                                      jnp.bfloat16).astype(jnp.float32)
    hi = jax.lax.bitcast_convert_type((u >> 16).astype(jnp.uint16),
                                      jnp.bfloat16).astype(jnp.float32)
    return lo, hi


def _proj_body(x_ref, al_ref, ah_ref, bl_ref, bh_ref, bel_ref, beh_ref,
               tab_ref):
    x = x_ref[...]
    xal = jnp.dot(x, al_ref[...], preferred_element_type=jnp.float32) + bel_ref[...]
    xah = jnp.dot(x, ah_ref[...], preferred_element_type=jnp.float32) + beh_ref[...]
    xbl = jnp.dot(x, bl_ref[...], preferred_element_type=jnp.float32)
    xbh = jnp.dot(x, bh_ref[...], preferred_element_type=jnp.float32)
    tab_ref[...] = jnp.concatenate(
        [_pack16(xal, xah), _pack16(xbl, xbh)], axis=1)


def _proj(x, a, b, be1, rt=1000):
    n, h = x.shape
    hw = h // 2
    wspec = pl.BlockSpec((h, hw), lambda i: (0, 0))
    bspec = pl.BlockSpec((1, hw), lambda i: (0, 0))
    return pl.pallas_call(
        _proj_body,
        grid=(n // rt,),
        in_specs=[pl.BlockSpec((rt, h), lambda i: (i, 0)),
                  wspec, wspec, wspec, wspec, bspec, bspec],
        out_specs=pl.BlockSpec((rt, h), lambda i: (i, 0)),
        out_shape=jax.ShapeDtypeStruct((n, h), jnp.float32),
    )(x, a[:, :hw], a[:, hw:], b[:, :hw], b[:, hw:],
      be1[:, :hw], be1[:, hw:])


def _msg_body(g_ref, ea_ref, cl_ref, ch_ref, w2l_ref, w2h_ref,
              be2_ref, m2_ref):
    hw = g_ref.shape[1] // 2
    g = g_ref[...]
    gal, gah = _unpack16(g[:, :hw])
    gbl, gbh = _unpack16(g[:, hw:])
    ea = ea_ref[...]
    gl = gal + gbl + jnp.dot(ea, cl_ref[...], preferred_element_type=jnp.float32)
    gh = gah + gbh + jnp.dot(ea, ch_ref[...], preferred_element_type=jnp.float32)
    ml = _silu(gl)
    mh = _silu(gh)
    mm = (jnp.dot(ml, w2l_ref[...], preferred_element_type=jnp.float32)
          + jnp.dot(mh, w2h_ref[...], preferred_element_type=jnp.float32)
          + be2_ref[...])
    m2_ref[...] = _silu(mm)


def _msg(g, ea, c, we2, be2, et=2000):
    e, h = g.shape
    hw = h // 2
    de = ea.shape[1]
    return pl.pallas_call(
        _msg_body,
        grid=(e // et,),
        in_specs=[
            pl.BlockSpec((et, h), lambda i: (i, 0)),
            pl.BlockSpec((et, de), lambda i: (i, 0)),
            pl.BlockSpec((de, hw), lambda i: (0, 0)),
            pl.BlockSpec((de, hw), lambda i: (0, 0)),
            pl.BlockSpec((hw, h), lambda i: (0, 0)),
            pl.BlockSpec((hw, h), lambda i: (0, 0)),
            pl.BlockSpec((1, h), lambda i: (0, 0)),
        ],
        out_specs=pl.BlockSpec((et, h), lambda i: (i, 0)),
        out_shape=jax.ShapeDtypeStruct((e, h), jnp.float32),
    )(g, ea, c[:, :hw], c[:, hw:], we2[:hw], we2[hw:], be2)


def _node(x, aggs, wn1a, wn1b, bn1, wn2, bn2, rt=1000):
    n, h = x.shape
    na = len(aggs)

    def body(*refs):
        x_ref = refs[0]
        agg_refs = refs[1:1 + na]
        wn1a_ref, wn1b_ref, bn1_ref, wn2_ref, bn2_ref, o_ref = refs[1 + na:]
        x = x_ref[...]
        agg = sum(r[0] + r[1] for r in agg_refs)
        t = (jnp.dot(x, wn1a_ref[...], preferred_element_type=jnp.float32)
             + jnp.dot(agg, wn1b_ref[...], preferred_element_type=jnp.float32)
             + bn1_ref[...])
        t = _silu(t)
        o_ref[...] = x + jnp.dot(t, wn2_ref[...],
                                 preferred_element_type=jnp.float32) + bn2_ref[...]

    spec = pl.BlockSpec((rt, h), lambda i: (i, 0))
    aspec = pl.BlockSpec((2, rt, h), lambda i: (0, i, 0))
    wspec = pl.BlockSpec((h, h), lambda i: (0, 0))
    bspec = pl.BlockSpec((1, h), lambda i: (0, 0))
    return pl.pallas_call(
        body,
        grid=(n // rt,),
        in_specs=[spec] + [aspec] * na + [wspec, wspec, bspec, wspec, bspec],
        out_specs=spec,
        out_shape=jax.ShapeDtypeStruct((n, h), jnp.float32),
    )(x, *aggs, wn1a, wn1b, bn1, wn2, bn2)


def kernel(h, edges, edge_attr, params):
    n, d = h.shape
    hh = params['w_emb'].shape[1]
    row = edges[0]
    col = edges[1]
    zeros_nh = jnp.zeros((n, hh), jnp.float32)

    e = row.shape[0]
    eh = e // 2
    rows = (row[:eh], row[eh:])
    cols = (col[:eh], col[eh:])
    eas = (edge_attr[:eh], edge_attr[eh:])

    x = _embed(h, params['w_emb'], params['b_emb'].reshape(1, hh))
    for p in params['layers']:
        we1 = p['we1']
        a, b, c = we1[:hh], we1[hh:2 * hh], we1[2 * hh:]
        tab = _proj(x, a, b, p['be1'].reshape(1, hh))
        g = _sc_gather(tab, row, col)
        m2 = _msg(g, edge_attr, c, p['we2'], p['be2'].reshape(1, hh))
        agg = _sc_scatter(m2, row, zeros_nh)
        x = _node(x, (agg,), p['wn1'][:hh], p['wn1'][hh:],
                  p['bn1'].reshape(1, hh), p['wn2'], p['bn2'].reshape(1, hh))
    return _embed(x, params['w_out'], params['b_out'].reshape(1, d))


# speedup vs baseline: 1.1567x; 1.1567x over previous
import functools

import jax
import jax.numpy as jnp
from jax import lax
from jax.experimental import pallas as pl
from jax.experimental.pallas import tpu as pltpu
from jax.experimental.pallas import tpu_sc as plsc

NC = 2
NS = 16
NW = NC * NS
CHUNK = 128


def _mesh():
    return plsc.VectorSubcoreMesh(
        core_axis_name="c", subcore_axis_name="s", num_cores=NC, num_subcores=NS
    )


def _sc_gather(tab, row, col):
    n, h = tab.shape
    hw = h // 2
    e = row.shape[0]
    nch = e // CHUNK
    slots = -(-nch // NW)
    npairs = -(-slots // 2) * 2

    def body(tab_hbm, row_hbm, col_hbm, g_hbm, idxr, idxc, bufr, bufc, sems):
        cid = lax.axis_index("c")
        sid = lax.axis_index("s")
        wid = sid * NC + cid

        def start(j, b):
            ch = j * NW + wid

            @pl.when(ch < nch)
            def _():
                base = ch * CHUNK
                pltpu.sync_copy(row_hbm.at[pl.ds(base, CHUNK)], idxr.at[b])
                pltpu.sync_copy(col_hbm.at[pl.ds(base, CHUNK)], idxc.at[b])
                pltpu.async_copy(tab_hbm.at[idxr.at[b]], bufr.at[b], sems.at[2 * b])
                pltpu.async_copy(tab_hbm.at[idxc.at[b]], bufc.at[b], sems.at[2 * b + 1])

        def drain(j, b):
            ch = j * NW + wid

            @pl.when(ch < nch)
            def _():
                base = ch * CHUNK
                pltpu.make_async_copy(tab_hbm.at[idxr.at[b]], bufr.at[b],
                                      sems.at[2 * b]).wait()
                pltpu.make_async_copy(tab_hbm.at[idxc.at[b]], bufc.at[b],
                                      sems.at[2 * b + 1]).wait()

                def splice(i, carry):
                    for k in range(hw // 16):
                        bufr[b, i, pl.ds(hw + k * 16, 16)] = (
                            bufc[b, i, pl.ds(hw + k * 16, 16)])
                    return carry

                lax.fori_loop(0, CHUNK, splice, 0)
                pltpu.sync_copy(bufr.at[b], g_hbm.at[pl.ds(base, CHUNK)])

        start(0, 0)

        def step(jj, carry):
            j0 = jj * 2
            start(j0 + 1, 1)
            drain(j0, 0)
            start(j0 + 2, 0)
            drain(j0 + 1, 1)
            return carry

        lax.fori_loop(0, npairs // 2, step, 0)

    f = pl.kernel(
        body,
        out_type=jax.ShapeDtypeStruct((e, h), jnp.float32),
        mesh=_mesh(),
        scratch_types=[
            pltpu.VMEM((2, CHUNK), jnp.int32),
            pltpu.VMEM((2, CHUNK), jnp.int32),
            pltpu.VMEM((2, CHUNK, h), jnp.float32),
            pltpu.VMEM((2, CHUNK, h), jnp.float32),
            pltpu.SemaphoreType.DMA((4,)),
        ],
    )
    return f(tab, row, col)


def _sc_scatter(m2, row, zeros_nh):
    e, h = m2.shape
    n = zeros_nh.shape[0]
    nch = e // CHUNK
    slots = -(-nch // NW)
    npairs = -(-slots // 2) * 2
    rq = (n // NS) // 8 * 8
    tail_off = NS * rq
    tail = n - tail_off

    def body(m2_hbm, row_hbm, z_hbm, agg_hbm, idx, buf, acc, sems):
        cid = lax.axis_index("c")
        sid = lax.axis_index("s")
        wid = sid * NC + cid

        pltpu.sync_copy(z_hbm.at[pl.ds(sid * rq, rq)],
                        acc.at[pl.ds(sid * rq, rq)])
        if tail:
            @pl.when(sid == 0)
            def _():
                pltpu.sync_copy(z_hbm.at[pl.ds(tail_off, tail)],
                                acc.at[pl.ds(tail_off, tail)])
        plsc.subcore_barrier()

        def load(j, b):
            ch = j * NW + wid

            @pl.when(ch < nch)
            def _():
                base = ch * CHUNK
                pltpu.sync_copy(row_hbm.at[pl.ds(base, CHUNK)], idx.at[b])
                pltpu.async_copy(m2_hbm.at[pl.ds(base, CHUNK)], buf.at[b],
                                 sems.at[b])

        def flush(j, b):
            ch = j * NW + wid

            @pl.when(ch < nch)
            def _():
                base = ch * CHUNK
                pltpu.make_async_copy(m2_hbm.at[pl.ds(base, CHUNK)],
                                      buf.at[b], sems.at[b]).wait()
                pltpu.sync_copy(buf.at[b], acc.at[idx.at[b]], add=True)

        load(0, 0)

        def step(jj, carry):
            j0 = jj * 2
            load(j0 + 1, 1)
            flush(j0, 0)
            load(j0 + 2, 0)
            flush(j0 + 1, 1)
            return carry

        lax.fori_loop(0, npairs // 2, step, 0)
        plsc.subcore_barrier()
        pltpu.sync_copy(acc.at[pl.ds(sid * rq, rq)],
                        agg_hbm.at[cid, pl.ds(sid * rq, rq)])
        if tail:
            @pl.when(sid == 0)
            def _():
                pltpu.sync_copy(acc.at[pl.ds(tail_off, tail)],
                                agg_hbm.at[cid, pl.ds(tail_off, tail)])

    f = pl.kernel(
        body,
        out_type=jax.ShapeDtypeStruct((NC, n, h), jnp.float32),
        mesh=_mesh(),
        scratch_types=[
            pltpu.VMEM((2, CHUNK), jnp.int32),
            pltpu.VMEM((2, CHUNK, h), jnp.float32),
            pltpu.VMEM_SHARED((n, h), jnp.float32),
            pltpu.SemaphoreType.DMA((2,)),
        ],
    )
    return f(m2, row, zeros_nh)


def _silu(x):
    return x * jax.nn.sigmoid(x)


def _row_specs(rt, h, n_weights, n_biases):
    in_specs = [pl.BlockSpec((rt, h), lambda i: (i, 0))]
    in_specs += [pl.BlockSpec((h, h), lambda i: (0, 0))] * n_weights
    in_specs += [pl.BlockSpec((1, h), lambda i: (0, 0))] * n_biases
    return in_specs


def _embed_body(h_ref, w_ref, b_ref, o_ref):
    o_ref[...] = jnp.dot(h_ref[...], w_ref[...],
                         preferred_element_type=jnp.float32) + b_ref[...]


def _embed(x, w, b, rt=1000):
    n, h = x.shape
    return pl.pallas_call(
        _embed_body,
        grid=(n // rt,),
        in_specs=_row_specs(rt, h, 1, 1),
        out_specs=pl.BlockSpec((rt, h), lambda i: (i, 0)),
        out_shape=jax.ShapeDtypeStruct((n, w.shape[1]), jnp.float32),
    )(x, w, b)


def _pack16(lo, hi):
    lo_u = jax.lax.bitcast_convert_type(lo.astype(jnp.bfloat16), jnp.uint16)
    hi_u = jax.lax.bitcast_convert_type(hi.astype(jnp.bfloat16), jnp.uint16)
    w = lo_u.astype(jnp.uint32) | (hi_u.astype(jnp.uint32) << 16)
    return jax.lax.bitcast_convert_type(w, jnp.float32)


def _unpack16(w):
    u = jax.lax.bitcast_convert_type(w, jnp.uint32)
    lo = jax.lax.bitcast_convert_type((u & 0xFFFF).astype(jnp.uint16),
                                      jnp.bfloat16).astype(jnp.float32)
    hi = jax.lax.bitcast_convert_type((u >> 16).astype(jnp.uint16),
                                      jnp.bfloat16).astype(jnp.float32)
    return lo, hi


def _proj_body(x_ref, al_ref, ah_ref, bl_ref, bh_ref, bel_ref, beh_ref,
               tab_ref):
    x = x_ref[...]
    xal = jnp.dot(x, al_ref[...], preferred_element_type=jnp.float32) + bel_ref[...]
    xah = jnp.dot(x, ah_ref[...], preferred_element_type=jnp.float32) + beh_ref[...]
    xbl = jnp.dot(x, bl_ref[...], preferred_element_type=jnp.float32)
    xbh = jnp.dot(x, bh_ref[...], preferred_element_type=jnp.float32)
    tab_ref[...] = jnp.concatenate(
        [_pack16(xal, xah), _pack16(xbl, xbh)], axis=1)


def _proj(x, a, b, be1, rt=1000):
    n, h = x.shape
    hw = h // 2
    wspec = pl.BlockSpec((h, hw), lambda i: (0, 0))
    bspec = pl.BlockSpec((1, hw), lambda i: (0, 0))
    return pl.pallas_call(
        _proj_body,
        grid=(n // rt,),
        in_specs=[pl.BlockSpec((rt, h), lambda i: (i, 0)),
                  wspec, wspec, wspec, wspec, bspec, bspec],
        out_specs=pl.BlockSpec((rt, h), lambda i: (i, 0)),
        out_shape=jax.ShapeDtypeStruct((n, h), jnp.float32),
    )(x, a[:, :hw], a[:, hw:], b[:, :hw], b[:, hw:],
      be1[:, :hw], be1[:, hw:])


def _msg_body(g_ref, ea_ref, cl_ref, ch_ref, w2l_ref, w2h_ref,
              be2_ref, m2_ref):
    hw = g_ref.shape[1] // 2
    g = g_ref[...]
    gal, gah = _unpack16(g[:, :hw])
    gbl, gbh = _unpack16(g[:, hw:])
    ea = ea_ref[...]
    gl = gal + gbl + jnp.dot(ea, cl_ref[...], preferred_element_type=jnp.float32)
    gh = gah + gbh + jnp.dot(ea, ch_ref[...], preferred_element_type=jnp.float32)
    ml = _silu(gl)
    mh = _silu(gh)
    mm = (jnp.dot(ml, w2l_ref[...], preferred_element_type=jnp.float32)
          + jnp.dot(mh, w2h_ref[...], preferred_element_type=jnp.float32)
          + be2_ref[...])
    m2_ref[...] = _silu(mm)


def _msg(g, ea, c, we2, be2, et=2000):
    e, h = g.shape
    hw = h // 2
    de = ea.shape[1]
    return pl.pallas_call(
        _msg_body,
        grid=(e // et,),
        in_specs=[
            pl.BlockSpec((et, h), lambda i: (i, 0)),
            pl.BlockSpec((et, de), lambda i: (i, 0)),
            pl.BlockSpec((de, hw), lambda i: (0, 0)),
            pl.BlockSpec((de, hw), lambda i: (0, 0)),
            pl.BlockSpec((hw, h), lambda i: (0, 0)),
            pl.BlockSpec((hw, h), lambda i: (0, 0)),
            pl.BlockSpec((1, h), lambda i: (0, 0)),
        ],
        out_specs=pl.BlockSpec((et, h), lambda i: (i, 0)),
        out_shape=jax.ShapeDtypeStruct((e, h), jnp.float32),
    )(g, ea, c[:, :hw], c[:, hw:], we2[:hw], we2[hw:], be2)


def _node(x, aggs, wn1a, wn1b, bn1, wn2, bn2, rt=1000):
    n, h = x.shape
    na = len(aggs)

    def body(*refs):
        x_ref = refs[0]
        agg_refs = refs[1:1 + na]
        wn1a_ref, wn1b_ref, bn1_ref, wn2_ref, bn2_ref, o_ref = refs[1 + na:]
        x = x_ref[...]
        agg = sum(r[0] + r[1] for r in agg_refs)
        t = (jnp.dot(x, wn1a_ref[...], preferred_element_type=jnp.float32)
             + jnp.dot(agg, wn1b_ref[...], preferred_element_type=jnp.float32)
             + bn1_ref[...])
        t = _silu(t)
        o_ref[...] = x + jnp.dot(t, wn2_ref[...],
                                 preferred_element_type=jnp.float32) + bn2_ref[...]

    spec = pl.BlockSpec((rt, h), lambda i: (i, 0))
    aspec = pl.BlockSpec((2, rt, h), lambda i: (0, i, 0))
    wspec = pl.BlockSpec((h, h), lambda i: (0, 0))
    bspec = pl.BlockSpec((1, h), lambda i: (0, 0))
    return pl.pallas_call(
        body,
        grid=(n // rt,),
        in_specs=[spec] + [aspec] * na + [wspec, wspec, bspec, wspec, bspec],
        out_specs=spec,
        out_shape=jax.ShapeDtypeStruct((n, h), jnp.float32),
    )(x, *aggs, wn1a, wn1b, bn1, wn2, bn2)


def kernel(h, edges, edge_attr, params):
    n, d = h.shape
    hh = params['w_emb'].shape[1]
    row = edges[0]
    col = edges[1]
    zeros_nh = jnp.zeros((n, hh), jnp.float32)

    e = row.shape[0]
    nsplit = 2
    es = e // nsplit
    rows = tuple(row[k * es:(k + 1) * es] for k in range(nsplit))
    cols = tuple(col[k * es:(k + 1) * es] for k in range(nsplit))
    eas = tuple(edge_attr[k * es:(k + 1) * es] for k in range(nsplit))

    x = _embed(h, params['w_emb'], params['b_emb'].reshape(1, hh))
    for p in params['layers']:
        we1 = p['we1']
        a, b, c = we1[:hh], we1[hh:2 * hh], we1[2 * hh:]
        tab = _proj(x, a, b, p['be1'].reshape(1, hh))
        gs = [_sc_gather(tab, rows[k], cols[k]) for k in range(nsplit)]
        m2s = [_msg(gs[k], eas[k], c, p['we2'],
                    p['be2'].reshape(1, hh)) for k in range(nsplit)]
        aggs = tuple(_sc_scatter(m2s[k], rows[k], zeros_nh)
                     for k in range(nsplit))
        x = _node(x, aggs, p['wn1'][:hh], p['wn1'][hh:],
                  p['bn1'].reshape(1, hh), p['wn2'], p['bn2'].reshape(1, hh))
    return _embed(x, params['w_out'], params['b_out'].reshape(1, d))


# speedup vs baseline: 1.1760x; 1.0167x over previous
import functools

import jax
import jax.numpy as jnp
from jax import lax
from jax.experimental import pallas as pl
from jax.experimental.pallas import tpu as pltpu
from jax.experimental.pallas import tpu_sc as plsc

NC = 2
NS = 16
NW = NC * NS
CHUNK = 128


def _mesh():
    return plsc.VectorSubcoreMesh(
        core_axis_name="c", subcore_axis_name="s", num_cores=NC, num_subcores=NS
    )


def _sc_gather(tab, row, col):
    n, h = tab.shape
    hw = h // 2
    e = row.shape[0]
    nch = e // CHUNK
    slots = -(-nch // NW)
    npairs = -(-slots // 2) * 2

    def body(tab_hbm, row_hbm, col_hbm, g_hbm, idxr, idxc, bufr, bufc, sems):
        cid = lax.axis_index("c")
        sid = lax.axis_index("s")
        wid = sid * NC + cid

        def start(j, b):
            ch = j * NW + wid

            @pl.when(ch < nch)
            def _():
                base = ch * CHUNK
                pltpu.sync_copy(row_hbm.at[pl.ds(base, CHUNK)], idxr.at[b])
                pltpu.sync_copy(col_hbm.at[pl.ds(base, CHUNK)], idxc.at[b])
                pltpu.async_copy(tab_hbm.at[idxr.at[b]], bufr.at[b], sems.at[2 * b])
                pltpu.async_copy(tab_hbm.at[idxc.at[b]], bufc.at[b], sems.at[2 * b + 1])

        def drain(j, b):
            ch = j * NW + wid

            @pl.when(ch < nch)
            def _():
                base = ch * CHUNK
                pltpu.make_async_copy(tab_hbm.at[idxr.at[b]], bufr.at[b],
                                      sems.at[2 * b]).wait()
                pltpu.make_async_copy(tab_hbm.at[idxc.at[b]], bufc.at[b],
                                      sems.at[2 * b + 1]).wait()

                def splice(i, carry):
                    for k in range(hw // 16):
                        bufr[b, i, pl.ds(hw + k * 16, 16)] = (
                            bufc[b, i, pl.ds(hw + k * 16, 16)])
                    return carry

                lax.fori_loop(0, CHUNK, splice, 0)
                pltpu.sync_copy(bufr.at[b], g_hbm.at[pl.ds(base, CHUNK)])

        start(0, 0)

        def step(jj, carry):
            j0 = jj * 2
            start(j0 + 1, 1)
            drain(j0, 0)
            start(j0 + 2, 0)
            drain(j0 + 1, 1)
            return carry

        lax.fori_loop(0, npairs // 2, step, 0)

    f = pl.kernel(
        body,
        out_type=jax.ShapeDtypeStruct((e, h), jnp.float32),
        mesh=_mesh(),
        scratch_types=[
            pltpu.VMEM((2, CHUNK), jnp.int32),
            pltpu.VMEM((2, CHUNK), jnp.int32),
            pltpu.VMEM((2, CHUNK, h), jnp.float32),
            pltpu.VMEM((2, CHUNK, h), jnp.float32),
            pltpu.SemaphoreType.DMA((4,)),
        ],
    )
    return f(tab, row, col)


def _sc_scatter(m2, row, zeros_nh):
    e, h = m2.shape
    n = zeros_nh.shape[0]
    nch = e // CHUNK
    slots = -(-nch // NW)
    npairs = -(-slots // 2) * 2
    rq = (n // NS) // 8 * 8
    tail_off = NS * rq
    tail = n - tail_off

    def body(m2_hbm, row_hbm, z_hbm, agg_hbm, idx, buf, acc, sems):
        cid = lax.axis_index("c")
        sid = lax.axis_index("s")
        wid = sid * NC + cid

        pltpu.sync_copy(z_hbm.at[pl.ds(sid * rq, rq)],
                        acc.at[pl.ds(sid * rq, rq)])
        if tail:
            @pl.when(sid == 0)
            def _():
                pltpu.sync_copy(z_hbm.at[pl.ds(tail_off, tail)],
                                acc.at[pl.ds(tail_off, tail)])
        plsc.subcore_barrier()

        def load(j, b):
            ch = j * NW + wid

            @pl.when(ch < nch)
            def _():
                base = ch * CHUNK
                pltpu.sync_copy(row_hbm.at[pl.ds(base, CHUNK)], idx.at[b])
                pltpu.async_copy(m2_hbm.at[pl.ds(base, CHUNK)], buf.at[b],
                                 sems.at[b])

        def flush(j, b):
            ch = j * NW + wid

            @pl.when(ch < nch)
            def _():
                base = ch * CHUNK
                pltpu.make_async_copy(m2_hbm.at[pl.ds(base, CHUNK)],
                                      buf.at[b], sems.at[b]).wait()
                pltpu.sync_copy(buf.at[b], acc.at[idx.at[b]], add=True)

        load(0, 0)

        def step(jj, carry):
            j0 = jj * 2
            load(j0 + 1, 1)
            flush(j0, 0)
            load(j0 + 2, 0)
            flush(j0 + 1, 1)
            return carry

        lax.fori_loop(0, npairs // 2, step, 0)
        plsc.subcore_barrier()
        pltpu.sync_copy(acc.at[pl.ds(sid * rq, rq)],
                        agg_hbm.at[cid, pl.ds(sid * rq, rq)])
        if tail:
            @pl.when(sid == 0)
            def _():
                pltpu.sync_copy(acc.at[pl.ds(tail_off, tail)],
                                agg_hbm.at[cid, pl.ds(tail_off, tail)])

    f = pl.kernel(
        body,
        out_type=jax.ShapeDtypeStruct((NC, n, h), jnp.float32),
        mesh=_mesh(),
        scratch_types=[
            pltpu.VMEM((2, CHUNK), jnp.int32),
            pltpu.VMEM((2, CHUNK, h), jnp.float32),
            pltpu.VMEM_SHARED((n, h), jnp.float32),
            pltpu.SemaphoreType.DMA((2,)),
        ],
    )
    return f(m2, row, zeros_nh)


def _silu(x):
    return x * jax.nn.sigmoid(x)


def _row_specs(rt, h, n_weights, n_biases):
    in_specs = [pl.BlockSpec((rt, h), lambda i: (i, 0))]
    in_specs += [pl.BlockSpec((h, h), lambda i: (0, 0))] * n_weights
    in_specs += [pl.BlockSpec((1, h), lambda i: (0, 0))] * n_biases
    return in_specs


def _embed_body(h_ref, w_ref, b_ref, o_ref):
    o_ref[...] = jnp.dot(h_ref[...], w_ref[...],
                         preferred_element_type=jnp.float32) + b_ref[...]


def _embed(x, w, b, rt=1000):
    n, h = x.shape
    return pl.pallas_call(
        _embed_body,
        grid=(n // rt,),
        in_specs=_row_specs(rt, h, 1, 1),
        out_specs=pl.BlockSpec((rt, h), lambda i: (i, 0)),
        out_shape=jax.ShapeDtypeStruct((n, w.shape[1]), jnp.float32),
    )(x, w, b)


def _pack16(lo, hi):
    lo_u = jax.lax.bitcast_convert_type(lo.astype(jnp.bfloat16), jnp.uint16)
    hi_u = jax.lax.bitcast_convert_type(hi.astype(jnp.bfloat16), jnp.uint16)
    w = lo_u.astype(jnp.uint32) | (hi_u.astype(jnp.uint32) << 16)
    return jax.lax.bitcast_convert_type(w, jnp.float32)


def _unpack16(w):
    u = jax.lax.bitcast_convert_type(w, jnp.uint32)
    lo = jax.lax.bitcast_convert_type((u & 0xFFFF).astype(jnp.uint16),
                                      jnp.bfloat16).astype(jnp.float32)
    hi = jax.lax.bitcast_convert_type((u >> 16).astype(jnp.uint16),
                                      jnp.bfloat16).astype(jnp.float32)
    return lo, hi


def _proj_body(x_ref, al_ref, ah_ref, bl_ref, bh_ref, bel_ref, beh_ref,
               tab_ref):
    x = x_ref[...]
    xal = jnp.dot(x, al_ref[...], preferred_element_type=jnp.float32) + bel_ref[...]
    xah = jnp.dot(x, ah_ref[...], preferred_element_type=jnp.float32) + beh_ref[...]
    xbl = jnp.dot(x, bl_ref[...], preferred_element_type=jnp.float32)
    xbh = jnp.dot(x, bh_ref[...], preferred_element_type=jnp.float32)
    tab_ref[...] = jnp.concatenate(
        [_pack16(xal, xah), _pack16(xbl, xbh)], axis=1)


def _proj(x, a, b, be1, rt=1000):
    n, h = x.shape
    hw = h // 2
    wspec = pl.BlockSpec((h, hw), lambda i: (0, 0))
    bspec = pl.BlockSpec((1, hw), lambda i: (0, 0))
    return pl.pallas_call(
        _proj_body,
        grid=(n // rt,),
        in_specs=[pl.BlockSpec((rt, h), lambda i: (i, 0)),
                  wspec, wspec, wspec, wspec, bspec, bspec],
        out_specs=pl.BlockSpec((rt, h), lambda i: (i, 0)),
        out_shape=jax.ShapeDtypeStruct((n, h), jnp.float32),
    )(x, a[:, :hw], a[:, hw:], b[:, :hw], b[:, hw:],
      be1[:, :hw], be1[:, hw:])


def _msg_body(g_ref, ea_ref, cl_ref, ch_ref, w2l_ref, w2h_ref,
              be2_ref, m2_ref):
    hw = g_ref.shape[1] // 2
    g = g_ref[...]
    gal, gah = _unpack16(g[:, :hw])
    gbl, gbh = _unpack16(g[:, hw:])
    ea = ea_ref[...]
    gl = gal + gbl + jnp.dot(ea, cl_ref[...], preferred_element_type=jnp.float32)
    gh = gah + gbh + jnp.dot(ea, ch_ref[...], preferred_element_type=jnp.float32)
    ml = _silu(gl)
    mh = _silu(gh)
    mm = (jnp.dot(ml, w2l_ref[...], preferred_element_type=jnp.float32)
          + jnp.dot(mh, w2h_ref[...], preferred_element_type=jnp.float32)
          + be2_ref[...])
    m2_ref[...] = _silu(mm)


def _msg(g, ea, c, we2, be2, et=2000):
    e, h = g.shape
    hw = h // 2
    de = ea.shape[1]
    return pl.pallas_call(
        _msg_body,
        grid=(e // et,),
        in_specs=[
            pl.BlockSpec((et, h), lambda i: (i, 0)),
            pl.BlockSpec((et, de), lambda i: (i, 0)),
            pl.BlockSpec((de, hw), lambda i: (0, 0)),
            pl.BlockSpec((de, hw), lambda i: (0, 0)),
            pl.BlockSpec((hw, h), lambda i: (0, 0)),
            pl.BlockSpec((hw, h), lambda i: (0, 0)),
            pl.BlockSpec((1, h), lambda i: (0, 0)),
        ],
        out_specs=pl.BlockSpec((et, h), lambda i: (i, 0)),
        out_shape=jax.ShapeDtypeStruct((e, h), jnp.float32),
    )(g, ea, c[:, :hw], c[:, hw:], we2[:hw], we2[hw:], be2)


def _node(x, aggs, wn1a, wn1b, bn1, wn2, bn2, rt=1000):
    n, h = x.shape
    na = len(aggs)

    def body(*refs):
        x_ref = refs[0]
        agg_refs = refs[1:1 + na]
        wn1a_ref, wn1b_ref, bn1_ref, wn2_ref, bn2_ref, o_ref = refs[1 + na:]
        x = x_ref[...]
        agg = sum(r[0] + r[1] for r in agg_refs)
        t = (jnp.dot(x, wn1a_ref[...], preferred_element_type=jnp.float32)
             + jnp.dot(agg, wn1b_ref[...], preferred_element_type=jnp.float32)
             + bn1_ref[...])
        t = _silu(t)
        o_ref[...] = x + jnp.dot(t, wn2_ref[...],
                                 preferred_element_type=jnp.float32) + bn2_ref[...]

    spec = pl.BlockSpec((rt, h), lambda i: (i, 0))
    aspec = pl.BlockSpec((2, rt, h), lambda i: (0, i, 0))
    wspec = pl.BlockSpec((h, h), lambda i: (0, 0))
    bspec = pl.BlockSpec((1, h), lambda i: (0, 0))
    return pl.pallas_call(
        body,
        grid=(n // rt,),
        in_specs=[spec] + [aspec] * na + [wspec, wspec, bspec, wspec, bspec],
        out_specs=spec,
        out_shape=jax.ShapeDtypeStruct((n, h), jnp.float32),
    )(x, *aggs, wn1a, wn1b, bn1, wn2, bn2)


def kernel(h, edges, edge_attr, params):
    n, d = h.shape
    hh = params['w_emb'].shape[1]
    row = edges[0]
    col = edges[1]
    zeros_nh = jnp.zeros((n, hh), jnp.float32)

    e = row.shape[0]
    nsplit = 4
    es = e // nsplit
    rows = tuple(row[k * es:(k + 1) * es] for k in range(nsplit))
    cols = tuple(col[k * es:(k + 1) * es] for k in range(nsplit))
    eas = tuple(edge_attr[k * es:(k + 1) * es] for k in range(nsplit))

    x = _embed(h, params['w_emb'], params['b_emb'].reshape(1, hh))
    for p in params['layers']:
        we1 = p['we1']
        a, b, c = we1[:hh], we1[hh:2 * hh], we1[2 * hh:]
        tab = _proj(x, a, b, p['be1'].reshape(1, hh))
        gs = [_sc_gather(tab, rows[k], cols[k]) for k in range(nsplit)]
        m2s = [_msg(gs[k], eas[k], c, p['we2'],
                    p['be2'].reshape(1, hh)) for k in range(nsplit)]
        aggs = tuple(_sc_scatter(m2s[k], rows[k], zeros_nh)
                     for k in range(nsplit))
        x = _node(x, aggs, p['wn1'][:hh], p['wn1'][hh:],
                  p['bn1'].reshape(1, hh), p['wn2'], p['bn2'].reshape(1, hh))
    return _embed(x, params['w_out'], params['b_out'].reshape(1, d))
